# Initial kernel scaffold; baseline (speedup 1.0000x reference)
import jax
import jax.numpy as jnp
from jax.experimental import pallas as pl


def kernel(feat0, feat1, feat2, feat3, feat4, rpn_conv_w, rpn_conv_b, rpn_cls_w, rpn_cls_b, rpn_box_w, rpn_box_b, fc1_w, fc1_b, fc2_w, fc2_b, cls_w, cls_b, reg_w, reg_b, images_hw):
    raise NotImplementedError("write your pallas kernel here")


# speedup vs baseline: 3.5179x; 3.5179x over previous
import functools

import jax
import jax.numpy as jnp
from jax.experimental import pallas as pl
from jax.experimental.pallas import tpu as pltpu

STRIDES = (4, 8, 16, 32, 64)
RATIOS = (0.5, 1.0, 2.0)
ANCHOR_SCALE = 8.0
NUM_ANCHORS = 3
C_FEAT = 256
NUM_CLASSES = 80
IMG = 512
PRE_NMS_TOPK = 1000
POST_NMS_TOPK = 1000
NMS_THR = 0.7
ROI_OUT = 7
D_FC = 1024

NMS_BS = 128


def _conv(x, w, b):
    y = jax.lax.conv_general_dilated(
        x, w, (1, 1), 'SAME', dimension_numbers=('NCHW', 'OIHW', 'NCHW'))
    return y + b[None, :, None, None]


def _make_anchors(h, w, stride):
    r = jnp.asarray(RATIOS, jnp.float32)
    base = ANCHOR_SCALE * stride
    ws = base / jnp.sqrt(r)
    hs = base * jnp.sqrt(r)
    sx = (jnp.arange(w, dtype=jnp.float32) + 0.5) * stride
    sy = (jnp.arange(h, dtype=jnp.float32) + 0.5) * stride
    cy, cx = jnp.meshgrid(sy, sx, indexing='ij')
    cx = cx[:, :, None]
    cy = cy[:, :, None]
    anc = jnp.stack([cx - ws / 2, cy - hs / 2, cx + ws / 2, cy + hs / 2], axis=-1)
    return anc.reshape(-1, 4)


def _decode_clip(anchors, deltas):
    aw = anchors[:, 2] - anchors[:, 0]
    ah = anchors[:, 3] - anchors[:, 1]
    acx = anchors[:, 0] + 0.5 * aw
    acy = anchors[:, 1] + 0.5 * ah
    dx, dy, dw, dh = deltas[:, 0], deltas[:, 1], deltas[:, 2], deltas[:, 3]
    dw = jnp.clip(dw, -4.0, 4.0)
    dh = jnp.clip(dh, -4.0, 4.0)
    cx = acx + dx * aw
    cy = acy + dy * ah
    nw = aw * jnp.exp(dw)
    nh = ah * jnp.exp(dh)
    x1 = jnp.clip(cx - 0.5 * nw, 0.0, float(IMG))
    y1 = jnp.clip(cy - 0.5 * nh, 0.0, float(IMG))
    x2 = jnp.clip(cx + 0.5 * nw, 0.0, float(IMG))
    y2 = jnp.clip(cy + 0.5 * nh, 0.0, float(IMG))
    return jnp.stack([x1, y1, x2, y2], axis=-1)


def _nms_body(x1_ref, y1_ref, x2_ref, y2_ref, supp_ref, cnt_ref, *,
              nb, n_real):
    cnt_ref[0] = 0
    supp_ref[...] = jnp.zeros((nb, NMS_BS), jnp.float32)

    lane = jax.lax.broadcasted_iota(jnp.int32, (1, NMS_BS), 1)
    sub = jax.lax.broadcasted_iota(jnp.int32, (NMS_BS, NMS_BS), 0)
    lan2 = jax.lax.broadcasted_iota(jnp.int32, (NMS_BS, NMS_BS), 1)
    upper = sub < lan2

    def pair_iou(x1c, y1c, x2c, y2c, ac, x1r, y1r, x2r, y2r, ar):
        xx1 = jnp.maximum(x1c, x1r)
        yy1 = jnp.maximum(y1c, y1r)
        xx2 = jnp.minimum(x2c, x2r)
        yy2 = jnp.minimum(y2c, y2r)
        inter = jnp.maximum(xx2 - xx1, 0.0) * jnp.maximum(yy2 - yy1, 0.0)
        return inter / (ac + ar - inter + 1e-6)

    def area(x1, y1, x2, y2):
        return jnp.maximum(x2 - x1, 0.0) * jnp.maximum(y2 - y1, 0.0)

    def block(k, _):
        @pl.when(cnt_ref[0] < POST_NMS_TOPK)
        def _():
            x1r = x1_ref[pl.ds(k, 1), :]
            y1r = y1_ref[pl.ds(k, 1), :]
            x2r = x2_ref[pl.ds(k, 1), :]
            y2r = y2_ref[pl.ds(k, 1), :]
            ar = area(x1r, y1r, x2r, y2r)
            x1c = jnp.reshape(x1r, (NMS_BS, 1))
            y1c = jnp.reshape(y1r, (NMS_BS, 1))
            x2c = jnp.reshape(x2r, (NMS_BS, 1))
            y2c = jnp.reshape(y2r, (NMS_BS, 1))
            ac = jnp.reshape(ar, (NMS_BS, 1))

            m_kk = jnp.where(
                (pair_iou(x1c, y1c, x2c, y2c, ac, x1r, y1r, x2r, y2r, ar)
                 > NMS_THR) & upper, 1.0, 0.0)
            pre = supp_ref[pl.ds(k, 1), :]
            kept0 = 1.0 - pre

            def fix_cond(c):
                kept, prev, t = c
                return jnp.logical_and(
                    jnp.sum(jnp.abs(kept - prev)) > 0.0, t < NMS_BS + 2)

            def fix_body(c):
                kept, _, t = c
                kcol = jnp.reshape(kept, (NMS_BS, 1))
                hit = jnp.max(m_kk * kcol, axis=0, keepdims=True)
                new = jnp.where((pre > 0.5) | (hit > 0.5), 0.0, 1.0)
                return new, kept, t + 1

            kept, _, _ = jax.lax.while_loop(
                fix_cond, fix_body, (kept0, pre - 1.0, jnp.int32(0)))

            supp_ref[pl.ds(k, 1), :] = 1.0 - kept
            valid = (lane + k * NMS_BS) < n_real
            cnt_ref[0] = cnt_ref[0] + jnp.sum(
                jnp.where(valid, kept, 0.0).astype(jnp.int32))

            kcol = jnp.reshape(kept, (NMS_BS, 1))

            def later(j, _):
                x1j = x1_ref[pl.ds(j, 1), :]
                y1j = y1_ref[pl.ds(j, 1), :]
                x2j = x2_ref[pl.ds(j, 1), :]
                y2j = y2_ref[pl.ds(j, 1), :]
                aj = area(x1j, y1j, x2j, y2j)
                iou = pair_iou(x1c, y1c, x2c, y2c, ac, x1j, y1j, x2j, y2j, aj)
                hitm = jnp.where(iou > NMS_THR, 1.0, 0.0) * kcol
                hit = jnp.max(hitm, axis=0, keepdims=True)
                old = supp_ref[pl.ds(j, 1), :]
                supp_ref[pl.ds(j, 1), :] = jnp.maximum(old, hit)
                return 0

            jax.lax.fori_loop(k + 1, nb, later, 0)
        return 0

    jax.lax.fori_loop(0, nb, block, 0)


def _nms_pallas(boxes, scores):
    order = jnp.argsort(-scores)
    b = boxes[order]
    s = scores[order]
    n_real = b.shape[0]
    nb = -(-n_real // NMS_BS)
    n_pad = nb * NMS_BS
    bp = jnp.pad(b, ((0, n_pad - n_real), (0, 0)))
    planes = [bp[:, i].reshape(nb, NMS_BS) for i in range(4)]
    supp = pl.pallas_call(
        functools.partial(_nms_body, nb=nb, n_real=n_real),
        out_shape=jax.ShapeDtypeStruct((nb, NMS_BS), jnp.float32),
        scratch_shapes=[pltpu.SMEM((1,), jnp.int32)],
    )(*planes)
    suppb = supp.reshape(n_pad)[:n_real] > 0.5
    masked = jnp.where(suppb, -jnp.inf, s)
    topv, topi = jax.lax.top_k(masked, POST_NMS_TOPK)
    valid = jnp.isfinite(topv)
    out_b = jnp.where(valid[:, None], b[topi], 0.0)
    out_s = jnp.where(valid, topv, 0.0)
    return out_b, out_s


def _proposals(cls_list, box_list):
    all_b = []
    all_s = []
    for i in range(len(cls_list)):
        cls = cls_list[i]
        box = box_list[i]
        h, w = cls.shape[2], cls.shape[3]
        scores = jax.nn.sigmoid(cls[0].transpose(1, 2, 0).reshape(-1))
        deltas = box[0].reshape(NUM_ANCHORS, 4, h, w).transpose(2, 3, 0, 1).reshape(-1, 4)
        anchors = _make_anchors(h, w, STRIDES[i])
        k = min(PRE_NMS_TOPK, scores.shape[0])
        sv, si = jax.lax.top_k(scores, k)
        all_b.append(_decode_clip(anchors[si], deltas[si]))
        all_s.append(sv)
    return _nms_pallas(jnp.concatenate(all_b, 0), jnp.concatenate(all_s, 0))


def _roi_align_level(feat, boxes, stride):
    C, H, W = feat.shape
    N = boxes.shape[0]
    scale = 1.0 / stride
    x1 = boxes[:, 0] * scale
    y1 = boxes[:, 1] * scale
    x2 = boxes[:, 2] * scale
    y2 = boxes[:, 3] * scale
    bw = (x2 - x1) / ROI_OUT
    bh = (y2 - y1) / ROI_OUT
    g = jnp.arange(ROI_OUT, dtype=jnp.float32) + 0.5
    xs = x1[:, None] + g[None, :] * bw[:, None] - 0.5
    ys = y1[:, None] + g[None, :] * bh[:, None] - 0.5
    x0f = jnp.floor(xs)
    y0f = jnp.floor(ys)
    wx = xs - x0f
    wy = ys - y0f
    x0 = jnp.clip(x0f.astype(jnp.int32), 0, W - 1)
    x1i = jnp.clip(x0 + 1, 0, W - 1)
    y0 = jnp.clip(y0f.astype(jnp.int32), 0, H - 1)
    y1i = jnp.clip(y0 + 1, 0, H - 1)
    X0 = jnp.broadcast_to(x0[:, None, :], (N, ROI_OUT, ROI_OUT))
    X1 = jnp.broadcast_to(x1i[:, None, :], (N, ROI_OUT, ROI_OUT))
    Y0 = jnp.broadcast_to(y0[:, :, None], (N, ROI_OUT, ROI_OUT))
    Y1 = jnp.broadcast_to(y1i[:, :, None], (N, ROI_OUT, ROI_OUT))
    v00 = feat[:, Y0, X0]
    v01 = feat[:, Y0, X1]
    v10 = feat[:, Y1, X0]
    v11 = feat[:, Y1, X1]
    w00 = ((1 - wy)[:, :, None] * (1 - wx)[:, None, :])[None]
    w01 = ((1 - wy)[:, :, None] * wx[:, None, :])[None]
    w10 = (wy[:, :, None] * (1 - wx)[:, None, :])[None]
    w11 = (wy[:, :, None] * wx[:, None, :])[None]
    out = v00 * w00 + v01 * w01 + v10 * w10 + v11 * w11
    return out.transpose(1, 0, 2, 3)


_DOT = functools.partial(
    jax.lax.dot_general, precision=jax.lax.Precision.HIGHEST,
    preferred_element_type=jnp.float32)


def _fc1_body(x_ref, w_ref, b_ref, o_ref):
    @pl.when(pl.program_id(1) == 0)
    def _():
        o_ref[...] = jnp.zeros_like(o_ref)

    o_ref[...] += _DOT(x_ref[...], w_ref[...], (((1,), (0,)), ((), ())))

    @pl.when(pl.program_id(1) == pl.num_programs(1) - 1)
    def _():
        o_ref[...] = jnp.maximum(o_ref[...] + b_ref[...], 0.0)


def _head_body(y_ref, w2_ref, b2_ref, cw_ref, cb_ref, rw_ref, rb_ref,
               cls_ref, reg_ref):
    h = jnp.maximum(
        _DOT(y_ref[...], w2_ref[...], (((1,), (0,)), ((), ()))) + b2_ref[...],
        0.0)
    cls_ref[...] = _DOT(h, cw_ref[...], (((1,), (0,)), ((), ()))) + cb_ref[...]
    reg_ref[...] = _DOT(h, rw_ref[...], (((1,), (0,)), ((), ()))) + rb_ref[...]


def _fc_head(x, fc1_w, fc1_b, fc2_w, fc2_b, cls_w, cls_b, reg_w, reg_b):
    n = x.shape[0]
    npad = 1024
    d_in = x.shape[1]
    xp = jnp.pad(x, ((0, npad - n), (0, 0)))
    mb, kb = 256, 1792
    nk = d_in // kb
    y1 = pl.pallas_call(
        _fc1_body,
        grid=(npad // mb, nk),
        in_specs=[
            pl.BlockSpec((mb, kb), lambda m, k: (m, k)),
            pl.BlockSpec((kb, D_FC), lambda m, k: (k, 0)),
            pl.BlockSpec((1, D_FC), lambda m, k: (0, 0)),
        ],
        out_specs=pl.BlockSpec((mb, D_FC), lambda m, k: (m, 0)),
        out_shape=jax.ShapeDtypeStruct((npad, D_FC), jnp.float32),
    )(xp, fc1_w, fc1_b.reshape(1, D_FC))

    cls, reg = pl.pallas_call(
        _head_body,
        grid=(npad // mb,),
        in_specs=[
            pl.BlockSpec((mb, D_FC), lambda m: (m, 0)),
            pl.BlockSpec((D_FC, D_FC), lambda m: (0, 0)),
            pl.BlockSpec((1, D_FC), lambda m: (0, 0)),
            pl.BlockSpec((D_FC, NUM_CLASSES + 1), lambda m: (0, 0)),
            pl.BlockSpec((1, NUM_CLASSES + 1), lambda m: (0, 0)),
            pl.BlockSpec((D_FC, NUM_CLASSES * 4), lambda m: (0, 0)),
            pl.BlockSpec((1, NUM_CLASSES * 4), lambda m: (0, 0)),
        ],
        out_specs=[
            pl.BlockSpec((mb, NUM_CLASSES + 1), lambda m: (m, 0)),
            pl.BlockSpec((mb, NUM_CLASSES * 4), lambda m: (m, 0)),
        ],
        out_shape=[
            jax.ShapeDtypeStruct((npad, NUM_CLASSES + 1), jnp.float32),
            jax.ShapeDtypeStruct((npad, NUM_CLASSES * 4), jnp.float32),
        ],
    )(y1, fc2_w, fc2_b.reshape(1, D_FC), cls_w,
      cls_b.reshape(1, NUM_CLASSES + 1), reg_w,
      reg_b.reshape(1, NUM_CLASSES * 4))
    return cls[:n], reg[:n]


def _roi_head(feats, boxes, fc1_w, fc1_b, fc2_w, fc2_b, cls_w, cls_b,
              reg_w, reg_b):
    area = (boxes[:, 2] - boxes[:, 0]) * (boxes[:, 3] - boxes[:, 1])
    lvl = jnp.floor(4.0 + jnp.log2(jnp.sqrt(jnp.maximum(area, 1e-6)) / 224.0))
    lvl = jnp.clip(lvl, 2.0, 5.0).astype(jnp.int32) - 2
    N = boxes.shape[0]
    pooled = jnp.zeros((N, C_FEAT, ROI_OUT, ROI_OUT), jnp.float32)
    for li in range(4):
        p = _roi_align_level(feats[li][0], boxes, STRIDES[li])
        pooled = pooled + p * (lvl == li)[:, None, None, None].astype(jnp.float32)
    x = pooled.reshape(N, -1)
    return _fc_head(x, fc1_w, fc1_b, fc2_w, fc2_b, cls_w, cls_b, reg_w, reg_b)


def kernel(feat0, feat1, feat2, feat3, feat4, rpn_conv_w, rpn_conv_b,
           rpn_cls_w, rpn_cls_b, rpn_box_w, rpn_box_b, fc1_w, fc1_b,
           fc2_w, fc2_b, cls_w, cls_b, reg_w, reg_b, images_hw):
    feats = [feat0, feat1, feat2, feat3, feat4]
    cls_list = []
    box_list = []
    for f in feats:
        h = jax.nn.relu(_conv(f, rpn_conv_w, rpn_conv_b))
        cls_list.append(_conv(h, rpn_cls_w, rpn_cls_b))
        box_list.append(_conv(h, rpn_box_w, rpn_box_b))
    prop_boxes, prop_scores = _proposals(cls_list, box_list)
    roi_cls, roi_reg = _roi_head(feats, prop_boxes, fc1_w, fc1_b, fc2_w,
                                 fc2_b, cls_w, cls_b, reg_w, reg_b)
    return (roi_cls, roi_reg, prop_boxes, prop_scores)


# speedup vs baseline: 29.0647x; 8.2619x over previous
import functools

import jax
import jax.numpy as jnp
from jax import lax
from jax.experimental import pallas as pl
from jax.experimental.pallas import tpu as pltpu
from jax.experimental.pallas import tpu_sc as plsc

STRIDES = (4, 8, 16, 32, 64)
RATIOS = (0.5, 1.0, 2.0)
ANCHOR_SCALE = 8.0
NUM_ANCHORS = 3
C_FEAT = 256
NUM_CLASSES = 80
IMG = 512
PRE_NMS_TOPK = 1000
POST_NMS_TOPK = 1000
NMS_THR = 0.7
ROI_OUT = 7
D_FC = 1024

NMS_BS = 128


def _conv(x, w, b):
    y = jax.lax.conv_general_dilated(
        x, w, (1, 1), 'SAME', dimension_numbers=('NCHW', 'OIHW', 'NCHW'))
    return y + b[None, :, None, None]


def _make_anchors(h, w, stride):
    r = jnp.asarray(RATIOS, jnp.float32)
    base = ANCHOR_SCALE * stride
    ws = base / jnp.sqrt(r)
    hs = base * jnp.sqrt(r)
    sx = (jnp.arange(w, dtype=jnp.float32) + 0.5) * stride
    sy = (jnp.arange(h, dtype=jnp.float32) + 0.5) * stride
    cy, cx = jnp.meshgrid(sy, sx, indexing='ij')
    cx = cx[:, :, None]
    cy = cy[:, :, None]
    anc = jnp.stack([cx - ws / 2, cy - hs / 2, cx + ws / 2, cy + hs / 2], axis=-1)
    return anc.reshape(-1, 4)


def _decode_clip(anchors, deltas):
    aw = anchors[:, 2] - anchors[:, 0]
    ah = anchors[:, 3] - anchors[:, 1]
    acx = anchors[:, 0] + 0.5 * aw
    acy = anchors[:, 1] + 0.5 * ah
    dx, dy, dw, dh = deltas[:, 0], deltas[:, 1], deltas[:, 2], deltas[:, 3]
    dw = jnp.clip(dw, -4.0, 4.0)
    dh = jnp.clip(dh, -4.0, 4.0)
    cx = acx + dx * aw
    cy = acy + dy * ah
    nw = aw * jnp.exp(dw)
    nh = ah * jnp.exp(dh)
    x1 = jnp.clip(cx - 0.5 * nw, 0.0, float(IMG))
    y1 = jnp.clip(cy - 0.5 * nh, 0.0, float(IMG))
    x2 = jnp.clip(cx + 0.5 * nw, 0.0, float(IMG))
    y2 = jnp.clip(cy + 0.5 * nh, 0.0, float(IMG))
    return jnp.stack([x1, y1, x2, y2], axis=-1)


def _nms_body(x1_ref, y1_ref, x2_ref, y2_ref, supp_ref, cnt_ref, *,
              nb, n_real):
    cnt_ref[0] = 0
    supp_ref[...] = jnp.zeros((nb, NMS_BS), jnp.float32)

    lane = jax.lax.broadcasted_iota(jnp.int32, (1, NMS_BS), 1)
    sub = jax.lax.broadcasted_iota(jnp.int32, (NMS_BS, NMS_BS), 0)
    lan2 = jax.lax.broadcasted_iota(jnp.int32, (NMS_BS, NMS_BS), 1)
    upper = sub < lan2

    def pair_iou(x1c, y1c, x2c, y2c, ac, x1r, y1r, x2r, y2r, ar):
        xx1 = jnp.maximum(x1c, x1r)
        yy1 = jnp.maximum(y1c, y1r)
        xx2 = jnp.minimum(x2c, x2r)
        yy2 = jnp.minimum(y2c, y2r)
        inter = jnp.maximum(xx2 - xx1, 0.0) * jnp.maximum(yy2 - yy1, 0.0)
        return inter / (ac + ar - inter + 1e-6)

    def area(x1, y1, x2, y2):
        return jnp.maximum(x2 - x1, 0.0) * jnp.maximum(y2 - y1, 0.0)

    def block(k, _):
        @pl.when(cnt_ref[0] < POST_NMS_TOPK)
        def _():
            x1r = x1_ref[pl.ds(k, 1), :]
            y1r = y1_ref[pl.ds(k, 1), :]
            x2r = x2_ref[pl.ds(k, 1), :]
            y2r = y2_ref[pl.ds(k, 1), :]
            ar = area(x1r, y1r, x2r, y2r)
            x1c = jnp.reshape(x1r, (NMS_BS, 1))
            y1c = jnp.reshape(y1r, (NMS_BS, 1))
            x2c = jnp.reshape(x2r, (NMS_BS, 1))
            y2c = jnp.reshape(y2r, (NMS_BS, 1))
            ac = jnp.reshape(ar, (NMS_BS, 1))

            m_kk = jnp.where(
                (pair_iou(x1c, y1c, x2c, y2c, ac, x1r, y1r, x2r, y2r, ar)
                 > NMS_THR) & upper, 1.0, 0.0)
            pre = supp_ref[pl.ds(k, 1), :]
            kept0 = 1.0 - pre

            def fix_cond(c):
                kept, prev, t = c
                return jnp.logical_and(
                    jnp.sum(jnp.abs(kept - prev)) > 0.0, t < NMS_BS + 2)

            def fix_body(c):
                kept, _, t = c
                kcol = jnp.reshape(kept, (NMS_BS, 1))
                hit = jnp.max(m_kk * kcol, axis=0, keepdims=True)
                new = jnp.where((pre > 0.5) | (hit > 0.5), 0.0, 1.0)
                return new, kept, t + 1

            kept, _, _ = jax.lax.while_loop(
                fix_cond, fix_body, (kept0, pre - 1.0, jnp.int32(0)))

            supp_ref[pl.ds(k, 1), :] = 1.0 - kept
            valid = (lane + k * NMS_BS) < n_real
            cnt_ref[0] = cnt_ref[0] + jnp.sum(
                jnp.where(valid, kept, 0.0).astype(jnp.int32))

            kcol = jnp.reshape(kept, (NMS_BS, 1))

            def later(j, _):
                x1j = x1_ref[pl.ds(j, 1), :]
                y1j = y1_ref[pl.ds(j, 1), :]
                x2j = x2_ref[pl.ds(j, 1), :]
                y2j = y2_ref[pl.ds(j, 1), :]
                aj = area(x1j, y1j, x2j, y2j)
                iou = pair_iou(x1c, y1c, x2c, y2c, ac, x1j, y1j, x2j, y2j, aj)
                hitm = jnp.where(iou > NMS_THR, 1.0, 0.0) * kcol
                hit = jnp.max(hitm, axis=0, keepdims=True)
                old = supp_ref[pl.ds(j, 1), :]
                supp_ref[pl.ds(j, 1), :] = jnp.maximum(old, hit)
                return 0

            jax.lax.fori_loop(k + 1, nb, later, 0)
        return 0

    jax.lax.fori_loop(0, nb, block, 0)


def _nms_pallas(boxes, scores):
    order = jnp.argsort(-scores)
    b = boxes[order]
    s = scores[order]
    n_real = b.shape[0]
    nb = -(-n_real // NMS_BS)
    n_pad = nb * NMS_BS
    bp = jnp.pad(b, ((0, n_pad - n_real), (0, 0)))
    planes = [bp[:, i].reshape(nb, NMS_BS) for i in range(4)]
    supp = pl.pallas_call(
        functools.partial(_nms_body, nb=nb, n_real=n_real),
        out_shape=jax.ShapeDtypeStruct((nb, NMS_BS), jnp.float32),
        scratch_shapes=[pltpu.SMEM((1,), jnp.int32)],
    )(*planes)
    suppb = supp.reshape(n_pad)[:n_real] > 0.5
    masked = jnp.where(suppb, -jnp.inf, s)
    topv, topi = jax.lax.top_k(masked, POST_NMS_TOPK)
    valid = jnp.isfinite(topv)
    out_b = jnp.where(valid[:, None], b[topi], 0.0)
    out_s = jnp.where(valid, topv, 0.0)
    return out_b, out_s


def _proposals(cls_list, box_list):
    all_b = []
    all_s = []
    for i in range(len(cls_list)):
        cls = cls_list[i]
        box = box_list[i]
        h, w = cls.shape[2], cls.shape[3]
        scores = jax.nn.sigmoid(cls[0].transpose(1, 2, 0).reshape(-1))
        deltas = box[0].reshape(NUM_ANCHORS, 4, h, w).transpose(2, 3, 0, 1).reshape(-1, 4)
        anchors = _make_anchors(h, w, STRIDES[i])
        k = min(PRE_NMS_TOPK, scores.shape[0])
        sv, si = jax.lax.top_k(scores, k)
        all_b.append(_decode_clip(anchors[si], deltas[si]))
        all_s.append(sv)
    return _nms_pallas(jnp.concatenate(all_b, 0), jnp.concatenate(all_s, 0))


ROI_PTS = 1000 * ROI_OUT * ROI_OUT
ROI_PTS_PAD = 49152
SC_CHUNK = 128


def _sc_gather(table, idx):
    info = plsc.get_sparse_core_info()
    nw = info.num_cores * info.num_subcores
    m = idx.shape[0]
    per_w = m // nw
    nch = per_w // SC_CHUNK
    mesh = plsc.VectorSubcoreMesh(core_axis_name="c", subcore_axis_name="s")

    @functools.partial(
        pl.kernel, mesh=mesh,
        out_type=jax.ShapeDtypeStruct((m, C_FEAT), jnp.float32),
        scratch_types=[
            pltpu.VMEM((SC_CHUNK,), jnp.int32),
            pltpu.VMEM((SC_CHUNK, C_FEAT), jnp.float32),
            pltpu.SemaphoreType.DMA,
        ],
    )
    def k(table_hbm, idx_hbm, out_hbm, idx_v, rows_v, sem):
        wid = lax.axis_index("s") * info.num_cores + lax.axis_index("c")
        base = wid * per_w

        def chunk(i, _):
            off = base + i * SC_CHUNK
            pltpu.sync_copy(idx_hbm.at[pl.ds(off, SC_CHUNK)], idx_v)
            pltpu.async_copy(table_hbm.at[idx_v], rows_v, sem).wait()
            pltpu.sync_copy(rows_v, out_hbm.at[pl.ds(off, SC_CHUNK)])
            return 0

        jax.lax.fori_loop(0, nch, chunk, 0)

    return k(table, idx)


def _blend_body(r00, r01, r10, r11, w00, w01, w10, w11, o_ref):
    o_ref[...] = (r00[...] * w00[...] + r01[...] * w01[...]
                  + r10[...] * w10[...] + r11[...] * w11[...])


def _roi_align_sc(feats, boxes):
    n = boxes.shape[0]
    tabs = []
    offs = []
    off = 0
    for li in range(4):
        f = feats[li][0]
        c, h, w = f.shape
        tabs.append(f.reshape(c, h * w).T)
        offs.append(off)
        off += h * w
    table = jnp.concatenate(tabs, axis=0)

    area = (boxes[:, 2] - boxes[:, 0]) * (boxes[:, 3] - boxes[:, 1])
    lvl = jnp.floor(4.0 + jnp.log2(jnp.sqrt(jnp.maximum(area, 1e-6)) / 224.0))
    lvl = jnp.clip(lvl, 2.0, 5.0).astype(jnp.int32) - 2

    strides = jnp.asarray(STRIDES[:4], jnp.float32)[lvl]
    wls = jnp.asarray([IMG // s for s in STRIDES[:4]], jnp.int32)[lvl]
    offsets = jnp.asarray(offs, jnp.int32)[lvl]

    scale = 1.0 / strides
    x1 = boxes[:, 0] * scale
    y1 = boxes[:, 1] * scale
    x2 = boxes[:, 2] * scale
    y2 = boxes[:, 3] * scale
    bw = (x2 - x1) / ROI_OUT
    bh = (y2 - y1) / ROI_OUT
    g = jnp.arange(ROI_OUT, dtype=jnp.float32) + 0.5
    xs = x1[:, None] + g[None, :] * bw[:, None] - 0.5
    ys = y1[:, None] + g[None, :] * bh[:, None] - 0.5
    x0f = jnp.floor(xs)
    y0f = jnp.floor(ys)
    wx = xs - x0f
    wy = ys - y0f
    wl1 = wls - 1
    x0 = jnp.clip(x0f.astype(jnp.int32), 0, wl1[:, None])
    x1i = jnp.clip(x0 + 1, 0, wl1[:, None])
    y0 = jnp.clip(y0f.astype(jnp.int32), 0, wl1[:, None])
    y1i = jnp.clip(y0 + 1, 0, wl1[:, None])

    def flat_idx(yy, xx):
        r = offsets[:, None, None] + yy[:, :, None] * wls[:, None, None] \
            + xx[:, None, :]
        return jnp.pad(r.reshape(-1), (0, ROI_PTS_PAD - ROI_PTS))

    idx_all = jnp.concatenate(
        [flat_idx(y0, x0), flat_idx(y0, x1i),
         flat_idx(y1i, x0), flat_idx(y1i, x1i)], axis=0)

    rows = _sc_gather(table, idx_all)

    def wgt(a, b):
        w = (a[:, :, None] * b[:, None, :]).reshape(-1)
        return jnp.pad(w, (0, ROI_PTS_PAD - ROI_PTS)).reshape(ROI_PTS_PAD, 1)

    w00 = wgt(1 - wy, 1 - wx)
    w01 = wgt(1 - wy, wx)
    w10 = wgt(wy, 1 - wx)
    w11 = wgt(wy, wx)

    mbb = 1536
    blended = pl.pallas_call(
        _blend_body,
        grid=(ROI_PTS_PAD // mbb,),
        in_specs=(
            [pl.BlockSpec((mbb, C_FEAT), lambda i, c=c: (i + c * (ROI_PTS_PAD // mbb), 0))
             for c in range(4)]
            + [pl.BlockSpec((mbb, 1), lambda i: (i, 0))] * 4),
        out_specs=pl.BlockSpec((mbb, C_FEAT), lambda i: (i, 0)),
        out_shape=jax.ShapeDtypeStruct((ROI_PTS_PAD, C_FEAT), jnp.float32),
    )(rows, rows, rows, rows, w00, w01, w10, w11)
    return blended[:ROI_PTS].reshape(n, ROI_OUT * ROI_OUT * C_FEAT)


_DOT = functools.partial(
    jax.lax.dot_general, precision=jax.lax.Precision.HIGHEST,
    preferred_element_type=jnp.float32)


def _fc1_body(x_ref, w_ref, b_ref, o_ref):
    @pl.when(pl.program_id(1) == 0)
    def _():
        o_ref[...] = jnp.zeros_like(o_ref)

    o_ref[...] += _DOT(x_ref[...], w_ref[...], (((1,), (0,)), ((), ())))

    @pl.when(pl.program_id(1) == pl.num_programs(1) - 1)
    def _():
        o_ref[...] = jnp.maximum(o_ref[...] + b_ref[...], 0.0)


def _head_body(y_ref, w2_ref, b2_ref, cw_ref, cb_ref, rw_ref, rb_ref,
               cls_ref, reg_ref):
    h = jnp.maximum(
        _DOT(y_ref[...], w2_ref[...], (((1,), (0,)), ((), ()))) + b2_ref[...],
        0.0)
    cls_ref[...] = _DOT(h, cw_ref[...], (((1,), (0,)), ((), ()))) + cb_ref[...]
    reg_ref[...] = _DOT(h, rw_ref[...], (((1,), (0,)), ((), ()))) + rb_ref[...]


def _fc_head(x, fc1_w, fc1_b, fc2_w, fc2_b, cls_w, cls_b, reg_w, reg_b):
    n = x.shape[0]
    npad = 1024
    d_in = x.shape[1]
    xp = jnp.pad(x, ((0, npad - n), (0, 0)))
    mb, kb = 256, 1792
    nk = d_in // kb
    y1 = pl.pallas_call(
        _fc1_body,
        grid=(npad // mb, nk),
        in_specs=[
            pl.BlockSpec((mb, kb), lambda m, k: (m, k)),
            pl.BlockSpec((kb, D_FC), lambda m, k: (k, 0)),
            pl.BlockSpec((1, D_FC), lambda m, k: (0, 0)),
        ],
        out_specs=pl.BlockSpec((mb, D_FC), lambda m, k: (m, 0)),
        out_shape=jax.ShapeDtypeStruct((npad, D_FC), jnp.float32),
    )(xp, fc1_w, fc1_b.reshape(1, D_FC))

    cls, reg = pl.pallas_call(
        _head_body,
        grid=(npad // mb,),
        in_specs=[
            pl.BlockSpec((mb, D_FC), lambda m: (m, 0)),
            pl.BlockSpec((D_FC, D_FC), lambda m: (0, 0)),
            pl.BlockSpec((1, D_FC), lambda m: (0, 0)),
            pl.BlockSpec((D_FC, NUM_CLASSES + 1), lambda m: (0, 0)),
            pl.BlockSpec((1, NUM_CLASSES + 1), lambda m: (0, 0)),
            pl.BlockSpec((D_FC, NUM_CLASSES * 4), lambda m: (0, 0)),
            pl.BlockSpec((1, NUM_CLASSES * 4), lambda m: (0, 0)),
        ],
        out_specs=[
            pl.BlockSpec((mb, NUM_CLASSES + 1), lambda m: (m, 0)),
            pl.BlockSpec((mb, NUM_CLASSES * 4), lambda m: (m, 0)),
        ],
        out_shape=[
            jax.ShapeDtypeStruct((npad, NUM_CLASSES + 1), jnp.float32),
            jax.ShapeDtypeStruct((npad, NUM_CLASSES * 4), jnp.float32),
        ],
    )(y1, fc2_w, fc2_b.reshape(1, D_FC), cls_w,
      cls_b.reshape(1, NUM_CLASSES + 1), reg_w,
      reg_b.reshape(1, NUM_CLASSES * 4))
    return cls[:n], reg[:n]


def _roi_head(feats, boxes, fc1_w, fc1_b, fc2_w, fc2_b, cls_w, cls_b,
              reg_w, reg_b):
    x = _roi_align_sc(feats, boxes)
    w1p = fc1_w.reshape(C_FEAT, ROI_OUT * ROI_OUT, D_FC).transpose(1, 0, 2) \
        .reshape(C_FEAT * ROI_OUT * ROI_OUT, D_FC)
    return _fc_head(x, w1p, fc1_b, fc2_w, fc2_b, cls_w, cls_b, reg_w, reg_b)


def kernel(feat0, feat1, feat2, feat3, feat4, rpn_conv_w, rpn_conv_b,
           rpn_cls_w, rpn_cls_b, rpn_box_w, rpn_box_b, fc1_w, fc1_b,
           fc2_w, fc2_b, cls_w, cls_b, reg_w, reg_b, images_hw):
    feats = [feat0, feat1, feat2, feat3, feat4]
    cls_list = []
    box_list = []
    for f in feats:
        h = jax.nn.relu(_conv(f, rpn_conv_w, rpn_conv_b))
        cls_list.append(_conv(h, rpn_cls_w, rpn_cls_b))
        box_list.append(_conv(h, rpn_box_w, rpn_box_b))
    prop_boxes, prop_scores = _proposals(cls_list, box_list)
    roi_cls, roi_reg = _roi_head(feats, prop_boxes, fc1_w, fc1_b, fc2_w,
                                 fc2_b, cls_w, cls_b, reg_w, reg_b)
    return (roi_cls, roi_reg, prop_boxes, prop_scores)


# speedup vs baseline: 29.3245x; 1.0089x over previous
import functools

import jax
import jax.numpy as jnp
from jax import lax
from jax.experimental import pallas as pl
from jax.experimental.pallas import tpu as pltpu
from jax.experimental.pallas import tpu_sc as plsc

STRIDES = (4, 8, 16, 32, 64)
RATIOS = (0.5, 1.0, 2.0)
ANCHOR_SCALE = 8.0
NUM_ANCHORS = 3
C_FEAT = 256
NUM_CLASSES = 80
IMG = 512
PRE_NMS_TOPK = 1000
POST_NMS_TOPK = 1000
NMS_THR = 0.7
ROI_OUT = 7
D_FC = 1024

NMS_BS = 128


def _conv(x, w, b):
    y = jax.lax.conv_general_dilated(
        x, w, (1, 1), 'SAME', dimension_numbers=('NCHW', 'OIHW', 'NCHW'))
    return y + b[None, :, None, None]


def _make_anchors(h, w, stride):
    r = jnp.asarray(RATIOS, jnp.float32)
    base = ANCHOR_SCALE * stride
    ws = base / jnp.sqrt(r)
    hs = base * jnp.sqrt(r)
    sx = (jnp.arange(w, dtype=jnp.float32) + 0.5) * stride
    sy = (jnp.arange(h, dtype=jnp.float32) + 0.5) * stride
    cy, cx = jnp.meshgrid(sy, sx, indexing='ij')
    cx = cx[:, :, None]
    cy = cy[:, :, None]
    anc = jnp.stack([cx - ws / 2, cy - hs / 2, cx + ws / 2, cy + hs / 2], axis=-1)
    return anc.reshape(-1, 4)


def _decode_clip(anchors, deltas):
    aw = anchors[:, 2] - anchors[:, 0]
    ah = anchors[:, 3] - anchors[:, 1]
    acx = anchors[:, 0] + 0.5 * aw
    acy = anchors[:, 1] + 0.5 * ah
    dx, dy, dw, dh = deltas[:, 0], deltas[:, 1], deltas[:, 2], deltas[:, 3]
    dw = jnp.clip(dw, -4.0, 4.0)
    dh = jnp.clip(dh, -4.0, 4.0)
    cx = acx + dx * aw
    cy = acy + dy * ah
    nw = aw * jnp.exp(dw)
    nh = ah * jnp.exp(dh)
    x1 = jnp.clip(cx - 0.5 * nw, 0.0, float(IMG))
    y1 = jnp.clip(cy - 0.5 * nh, 0.0, float(IMG))
    x2 = jnp.clip(cx + 0.5 * nw, 0.0, float(IMG))
    y2 = jnp.clip(cy + 0.5 * nh, 0.0, float(IMG))
    return jnp.stack([x1, y1, x2, y2], axis=-1)


def _nms_body(x1_ref, y1_ref, x2_ref, y2_ref, supp_ref, cnt_ref, *,
              nb, n_real):
    cnt_ref[0] = 0
    supp_ref[...] = jnp.zeros((nb, NMS_BS), jnp.float32)

    lane = jax.lax.broadcasted_iota(jnp.int32, (1, NMS_BS), 1)
    sub = jax.lax.broadcasted_iota(jnp.int32, (NMS_BS, NMS_BS), 0)
    lan2 = jax.lax.broadcasted_iota(jnp.int32, (NMS_BS, NMS_BS), 1)
    upper = sub < lan2

    def pair_iou(x1c, y1c, x2c, y2c, ac, x1r, y1r, x2r, y2r, ar):
        xx1 = jnp.maximum(x1c, x1r)
        yy1 = jnp.maximum(y1c, y1r)
        xx2 = jnp.minimum(x2c, x2r)
        yy2 = jnp.minimum(y2c, y2r)
        inter = jnp.maximum(xx2 - xx1, 0.0) * jnp.maximum(yy2 - yy1, 0.0)
        return inter / (ac + ar - inter + 1e-6)

    def area(x1, y1, x2, y2):
        return jnp.maximum(x2 - x1, 0.0) * jnp.maximum(y2 - y1, 0.0)

    def block(k, _):
        @pl.when(cnt_ref[0] < POST_NMS_TOPK)
        def _():
            x1r = x1_ref[pl.ds(k, 1), :]
            y1r = y1_ref[pl.ds(k, 1), :]
            x2r = x2_ref[pl.ds(k, 1), :]
            y2r = y2_ref[pl.ds(k, 1), :]
            ar = area(x1r, y1r, x2r, y2r)
            x1c = jnp.reshape(x1r, (NMS_BS, 1))
            y1c = jnp.reshape(y1r, (NMS_BS, 1))
            x2c = jnp.reshape(x2r, (NMS_BS, 1))
            y2c = jnp.reshape(y2r, (NMS_BS, 1))
            ac = jnp.reshape(ar, (NMS_BS, 1))

            m_kk = jnp.where(
                (pair_iou(x1c, y1c, x2c, y2c, ac, x1r, y1r, x2r, y2r, ar)
                 > NMS_THR) & upper, 1.0, 0.0)
            pre = supp_ref[pl.ds(k, 1), :]
            kept0 = 1.0 - pre

            def fix_cond(c):
                kept, prev, t = c
                return jnp.logical_and(
                    jnp.sum(jnp.abs(kept - prev)) > 0.0, t < NMS_BS + 2)

            def fix_body(c):
                kept, _, t = c
                kcol = jnp.reshape(kept, (NMS_BS, 1))
                hit = jnp.max(m_kk * kcol, axis=0, keepdims=True)
                new = jnp.where((pre > 0.5) | (hit > 0.5), 0.0, 1.0)
                return new, kept, t + 1

            kept, _, _ = jax.lax.while_loop(
                fix_cond, fix_body, (kept0, pre - 1.0, jnp.int32(0)))

            supp_ref[pl.ds(k, 1), :] = 1.0 - kept
            valid = (lane + k * NMS_BS) < n_real
            cnt_ref[0] = cnt_ref[0] + jnp.sum(
                jnp.where(valid, kept, 0.0).astype(jnp.int32))

            kcol = jnp.reshape(kept, (NMS_BS, 1))

            def later(j, _):
                x1j = x1_ref[pl.ds(j, 1), :]
                y1j = y1_ref[pl.ds(j, 1), :]
                x2j = x2_ref[pl.ds(j, 1), :]
                y2j = y2_ref[pl.ds(j, 1), :]
                aj = area(x1j, y1j, x2j, y2j)
                iou = pair_iou(x1c, y1c, x2c, y2c, ac, x1j, y1j, x2j, y2j, aj)
                hitm = jnp.where(iou > NMS_THR, 1.0, 0.0) * kcol
                hit = jnp.max(hitm, axis=0, keepdims=True)
                old = supp_ref[pl.ds(j, 1), :]
                supp_ref[pl.ds(j, 1), :] = jnp.maximum(old, hit)
                return 0

            jax.lax.fori_loop(k + 1, nb, later, 0)
        return 0

    jax.lax.fori_loop(0, nb, block, 0)


def _nms_pallas(boxes, scores):
    order = jnp.argsort(-scores)
    b = boxes[order]
    s = scores[order]
    n_real = b.shape[0]
    nb = -(-n_real // NMS_BS)
    n_pad = nb * NMS_BS
    bp = jnp.pad(b, ((0, n_pad - n_real), (0, 0)))
    planes = [bp[:, i].reshape(nb, NMS_BS) for i in range(4)]
    supp = pl.pallas_call(
        functools.partial(_nms_body, nb=nb, n_real=n_real),
        out_shape=jax.ShapeDtypeStruct((nb, NMS_BS), jnp.float32),
        scratch_shapes=[pltpu.SMEM((1,), jnp.int32)],
    )(*planes)
    suppb = supp.reshape(n_pad)[:n_real] > 0.5
    masked = jnp.where(suppb, -jnp.inf, s)
    topv, topi = jax.lax.top_k(masked, POST_NMS_TOPK)
    valid = jnp.isfinite(topv)
    out_b = jnp.where(valid[:, None], b[topi], 0.0)
    out_s = jnp.where(valid, topv, 0.0)
    return out_b, out_s


def _proposals(cls_list, box_list):
    all_b = []
    all_s = []
    for i in range(len(cls_list)):
        cls = cls_list[i]
        box = box_list[i]
        h, w = cls.shape[2], cls.shape[3]
        scores = jax.nn.sigmoid(cls[0].transpose(1, 2, 0).reshape(-1))
        deltas = box[0].reshape(NUM_ANCHORS, 4, h, w).transpose(2, 3, 0, 1).reshape(-1, 4)
        anchors = _make_anchors(h, w, STRIDES[i])
        k = min(PRE_NMS_TOPK, scores.shape[0])
        sv, si = jax.lax.top_k(scores, k)
        all_b.append(_decode_clip(anchors[si], deltas[si]))
        all_s.append(sv)
    return _nms_pallas(jnp.concatenate(all_b, 0), jnp.concatenate(all_s, 0))


ROI_PTS = 1000 * ROI_OUT * ROI_OUT
ROI_PTS_PAD = 49152
SC_CHUNK = 128


def _sc_gather(table, idx):
    info = plsc.get_sparse_core_info()
    nw = info.num_cores * info.num_subcores
    m = idx.shape[0]
    per_w = m // nw
    nch = per_w // SC_CHUNK
    mesh = plsc.VectorSubcoreMesh(core_axis_name="c", subcore_axis_name="s")

    nbuf = 3
    assert nch % nbuf == 0

    @functools.partial(
        pl.kernel, mesh=mesh,
        out_type=jax.ShapeDtypeStruct((m, C_FEAT), jnp.float32),
        scratch_types=[
            pltpu.VMEM((nbuf, SC_CHUNK), jnp.int32),
            pltpu.VMEM((nbuf, SC_CHUNK, C_FEAT), jnp.float32),
            [pltpu.SemaphoreType.DMA] * nbuf,
        ],
    )
    def k(table_hbm, idx_hbm, out_hbm, idx_v, rows_v, gsems):
        wid = lax.axis_index("s") * info.num_cores + lax.axis_index("c")
        base = wid * per_w

        def start_gather(ci, b):
            off = base + ci * SC_CHUNK
            pltpu.sync_copy(idx_hbm.at[pl.ds(off, SC_CHUNK)], idx_v.at[b])
            pltpu.async_copy(table_hbm.at[idx_v.at[b]], rows_v.at[b],
                             gsems[b])

        for b in range(nbuf):
            start_gather(b, b)

        def group(g, _):
            c0 = g * nbuf
            for b in range(nbuf):
                pltpu.make_async_copy(table_hbm.at[idx_v.at[b]],
                                      rows_v.at[b], gsems[b]).wait()
                pltpu.sync_copy(
                    rows_v.at[b],
                    out_hbm.at[pl.ds(base + (c0 + b) * SC_CHUNK, SC_CHUNK)])

                @pl.when(c0 + b + nbuf < nch)
                def _():
                    start_gather(c0 + b + nbuf, b)
            return 0

        jax.lax.fori_loop(0, nch // nbuf, group, 0)

    return k(table, idx)


def _blend_body(r00, r01, r10, r11, w00, w01, w10, w11, o_ref):
    o_ref[...] = (r00[...] * w00[...] + r01[...] * w01[...]
                  + r10[...] * w10[...] + r11[...] * w11[...])


def _roi_align_sc(feats, boxes):
    n = boxes.shape[0]
    tabs = []
    offs = []
    off = 0
    for li in range(4):
        f = feats[li][0]
        c, h, w = f.shape
        tabs.append(f.reshape(c, h * w).T)
        offs.append(off)
        off += h * w
    table = jnp.concatenate(tabs, axis=0)

    area = (boxes[:, 2] - boxes[:, 0]) * (boxes[:, 3] - boxes[:, 1])
    lvl = jnp.floor(4.0 + jnp.log2(jnp.sqrt(jnp.maximum(area, 1e-6)) / 224.0))
    lvl = jnp.clip(lvl, 2.0, 5.0).astype(jnp.int32) - 2

    strides = jnp.asarray(STRIDES[:4], jnp.float32)[lvl]
    wls = jnp.asarray([IMG // s for s in STRIDES[:4]], jnp.int32)[lvl]
    offsets = jnp.asarray(offs, jnp.int32)[lvl]

    scale = 1.0 / strides
    x1 = boxes[:, 0] * scale
    y1 = boxes[:, 1] * scale
    x2 = boxes[:, 2] * scale
    y2 = boxes[:, 3] * scale
    bw = (x2 - x1) / ROI_OUT
    bh = (y2 - y1) / ROI_OUT
    g = jnp.arange(ROI_OUT, dtype=jnp.float32) + 0.5
    xs = x1[:, None] + g[None, :] * bw[:, None] - 0.5
    ys = y1[:, None] + g[None, :] * bh[:, None] - 0.5
    x0f = jnp.floor(xs)
    y0f = jnp.floor(ys)
    wx = xs - x0f
    wy = ys - y0f
    wl1 = wls - 1
    x0 = jnp.clip(x0f.astype(jnp.int32), 0, wl1[:, None])
    x1i = jnp.clip(x0 + 1, 0, wl1[:, None])
    y0 = jnp.clip(y0f.astype(jnp.int32), 0, wl1[:, None])
    y1i = jnp.clip(y0 + 1, 0, wl1[:, None])

    def flat_idx(yy, xx):
        r = offsets[:, None, None] + yy[:, :, None] * wls[:, None, None] \
            + xx[:, None, :]
        return jnp.pad(r.reshape(-1), (0, ROI_PTS_PAD - ROI_PTS))

    idx_all = jnp.concatenate(
        [flat_idx(y0, x0), flat_idx(y0, x1i),
         flat_idx(y1i, x0), flat_idx(y1i, x1i)], axis=0)

    rows = _sc_gather(table, idx_all)

    def wgt(a, b):
        w = (a[:, :, None] * b[:, None, :]).reshape(-1)
        return jnp.pad(w, (0, ROI_PTS_PAD - ROI_PTS)).reshape(ROI_PTS_PAD, 1)

    w00 = wgt(1 - wy, 1 - wx)
    w01 = wgt(1 - wy, wx)
    w10 = wgt(wy, 1 - wx)
    w11 = wgt(wy, wx)

    mbb = 1536
    blended = pl.pallas_call(
        _blend_body,
        grid=(ROI_PTS_PAD // mbb,),
        in_specs=(
            [pl.BlockSpec((mbb, C_FEAT), lambda i, c=c: (i + c * (ROI_PTS_PAD // mbb), 0))
             for c in range(4)]
            + [pl.BlockSpec((mbb, 1), lambda i: (i, 0))] * 4),
        out_specs=pl.BlockSpec((mbb, C_FEAT), lambda i: (i, 0)),
        out_shape=jax.ShapeDtypeStruct((ROI_PTS_PAD, C_FEAT), jnp.float32),
    )(rows, rows, rows, rows, w00, w01, w10, w11)
    return blended[:ROI_PTS].reshape(n, ROI_OUT * ROI_OUT * C_FEAT)


_DOT = functools.partial(
    jax.lax.dot_general, precision=jax.lax.Precision.HIGHEST,
    preferred_element_type=jnp.float32)


def _fc1_body(x_ref, w_ref, b_ref, o_ref):
    @pl.when(pl.program_id(1) == 0)
    def _():
        o_ref[...] = jnp.zeros_like(o_ref)

    o_ref[...] += _DOT(x_ref[...], w_ref[...], (((1,), (0,)), ((), ())))

    @pl.when(pl.program_id(1) == pl.num_programs(1) - 1)
    def _():
        o_ref[...] = jnp.maximum(o_ref[...] + b_ref[...], 0.0)


def _head_body(y_ref, w2_ref, b2_ref, cw_ref, cb_ref, rw_ref, rb_ref,
               cls_ref, reg_ref):
    h = jnp.maximum(
        _DOT(y_ref[...], w2_ref[...], (((1,), (0,)), ((), ()))) + b2_ref[...],
        0.0)
    cls_ref[...] = _DOT(h, cw_ref[...], (((1,), (0,)), ((), ()))) + cb_ref[...]
    reg_ref[...] = _DOT(h, rw_ref[...], (((1,), (0,)), ((), ()))) + rb_ref[...]


def _fc_head(x, fc1_w, fc1_b, fc2_w, fc2_b, cls_w, cls_b, reg_w, reg_b):
    n = x.shape[0]
    npad = 1024
    d_in = x.shape[1]
    xp = jnp.pad(x, ((0, npad - n), (0, 0)))
    mb, kb = 256, 1792
    nk = d_in // kb
    y1 = pl.pallas_call(
        _fc1_body,
        grid=(npad // mb, nk),
        in_specs=[
            pl.BlockSpec((mb, kb), lambda m, k: (m, k)),
            pl.BlockSpec((kb, D_FC), lambda m, k: (k, 0)),
            pl.BlockSpec((1, D_FC), lambda m, k: (0, 0)),
        ],
        out_specs=pl.BlockSpec((mb, D_FC), lambda m, k: (m, 0)),
        out_shape=jax.ShapeDtypeStruct((npad, D_FC), jnp.float32),
    )(xp, fc1_w, fc1_b.reshape(1, D_FC))

    cls, reg = pl.pallas_call(
        _head_body,
        grid=(npad // mb,),
        in_specs=[
            pl.BlockSpec((mb, D_FC), lambda m: (m, 0)),
            pl.BlockSpec((D_FC, D_FC), lambda m: (0, 0)),
            pl.BlockSpec((1, D_FC), lambda m: (0, 0)),
            pl.BlockSpec((D_FC, NUM_CLASSES + 1), lambda m: (0, 0)),
            pl.BlockSpec((1, NUM_CLASSES + 1), lambda m: (0, 0)),
            pl.BlockSpec((D_FC, NUM_CLASSES * 4), lambda m: (0, 0)),
            pl.BlockSpec((1, NUM_CLASSES * 4), lambda m: (0, 0)),
        ],
        out_specs=[
            pl.BlockSpec((mb, NUM_CLASSES + 1), lambda m: (m, 0)),
            pl.BlockSpec((mb, NUM_CLASSES * 4), lambda m: (m, 0)),
        ],
        out_shape=[
            jax.ShapeDtypeStruct((npad, NUM_CLASSES + 1), jnp.float32),
            jax.ShapeDtypeStruct((npad, NUM_CLASSES * 4), jnp.float32),
        ],
    )(y1, fc2_w, fc2_b.reshape(1, D_FC), cls_w,
      cls_b.reshape(1, NUM_CLASSES + 1), reg_w,
      reg_b.reshape(1, NUM_CLASSES * 4))
    return cls[:n], reg[:n]


def _roi_head(feats, boxes, fc1_w, fc1_b, fc2_w, fc2_b, cls_w, cls_b,
              reg_w, reg_b):
    x = _roi_align_sc(feats, boxes)
    w1p = fc1_w.reshape(C_FEAT, ROI_OUT * ROI_OUT, D_FC).transpose(1, 0, 2) \
        .reshape(C_FEAT * ROI_OUT * ROI_OUT, D_FC)
    return _fc_head(x, w1p, fc1_b, fc2_w, fc2_b, cls_w, cls_b, reg_w, reg_b)


def kernel(feat0, feat1, feat2, feat3, feat4, rpn_conv_w, rpn_conv_b,
           rpn_cls_w, rpn_cls_b, rpn_box_w, rpn_box_b, fc1_w, fc1_b,
           fc2_w, fc2_b, cls_w, cls_b, reg_w, reg_b, images_hw):
    feats = [feat0, feat1, feat2, feat3, feat4]
    cls_list = []
    box_list = []
    for f in feats:
        h = jax.nn.relu(_conv(f, rpn_conv_w, rpn_conv_b))
        cls_list.append(_conv(h, rpn_cls_w, rpn_cls_b))
        box_list.append(_conv(h, rpn_box_w, rpn_box_b))
    prop_boxes, prop_scores = _proposals(cls_list, box_list)
    roi_cls, roi_reg = _roi_head(feats, prop_boxes, fc1_w, fc1_b, fc2_w,
                                 fc2_b, cls_w, cls_b, reg_w, reg_b)
    return (roi_cls, roi_reg, prop_boxes, prop_scores)
